# baseline (speedup 1.0000x reference)
import functools

import jax
import jax.numpy as jnp
from jax import lax
from jax.experimental import pallas as pl
from jax.experimental.pallas import tpu as pltpu
from jax.experimental.pallas import tpu_sc as plsc

S = 2048
D = 1024
FFN = 4096
E = 8
BT = 128
NB = (2 * S + E * (BT - 1) + BT - 1) // BT
BP = NB * BT
NA = 2 * S


def _router_body(x_ref, wg_ref, wfull_ref, sel_ref, rw_ref, aux_ref):
    x = x_ref[...]
    wg = wg_ref[...]
    logits = jnp.dot(x, wg, preferred_element_type=jnp.float32)
    pos = jax.lax.broadcasted_iota(jnp.int32, (S, E), 1)
    m1 = jnp.max(logits, axis=1, keepdims=True)
    p1 = jnp.min(jnp.where(logits == m1, pos, E), axis=1, keepdims=True)
    oh1 = pos == p1
    l2 = jnp.where(oh1, -jnp.inf, logits)
    m2 = jnp.max(l2, axis=1, keepdims=True)
    p2 = jnp.min(jnp.where(l2 == m2, pos, E), axis=1, keepdims=True)
    oh2 = pos == p2
    ed = jnp.exp(m2 - m1)
    w1 = 1.0 / (1.0 + ed)
    w2 = ed / (1.0 + ed)
    wfull_ref[...] = jnp.where(oh1, w1, 0.0) + jnp.where(oh2, w2, 0.0)
    zi = jnp.zeros((S, E - 2), jnp.int32)
    zf = jnp.zeros((S, E - 2), jnp.float32)
    sel_ref[...] = jnp.concatenate([p1, p2, zi], axis=1)
    rw_ref[...] = jnp.concatenate([w1, w2, zf], axis=1)
    sm = jnp.exp(logits - m1)
    sm = sm / jnp.sum(sm, axis=1, keepdims=True)
    prob = jnp.sum(sm, axis=0, keepdims=True) * (1.0 / S)
    tp = (jnp.sum(oh1.astype(jnp.float32), axis=0, keepdims=True)
          + jnp.sum(oh2.astype(jnp.float32), axis=0, keepdims=True)) * (1.0 / S)
    aux = jnp.sum(tp * prob) * E
    aux_ref[...] = jnp.full((8, 128), aux, jnp.float32)


def _sc_gather(table, idx, n_rows, n_chunks):
    info = plsc.get_sparse_core_info()
    nw = info.num_cores * info.num_subcores
    per_w = n_rows // nw
    chunk = per_w // n_chunks
    ncols = table.shape[1]
    dt = table.dtype
    mesh = plsc.VectorSubcoreMesh(core_axis_name="c", subcore_axis_name="s")

    @functools.partial(
        pl.kernel, mesh=mesh,
        out_type=jax.ShapeDtypeStruct((n_rows, ncols), dt),
        scratch_types=[
            pltpu.VMEM((per_w,), jnp.int32),
            pltpu.VMEM((chunk, ncols), dt),
            pltpu.VMEM((chunk, ncols), dt),
            pltpu.SemaphoreType.DMA,
            pltpu.SemaphoreType.DMA,
        ],
    )
    def k(table_hbm, idx_hbm, out_hbm, idx_v, rows0, rows1, sem0, sem1):
        wid = lax.axis_index("s") * info.num_cores + lax.axis_index("c")
        base = wid * per_w
        bufs = (rows0, rows1)
        sems = (sem0, sem1)
        pltpu.sync_copy(idx_hbm.at[pl.ds(base, per_w)], idx_v)
        hs = [None] * n_chunks
        hs[0] = pltpu.async_copy(
            table_hbm.at[idx_v.at[pl.ds(0, chunk)]], bufs[0], sems[0])
        for c in range(n_chunks):
            if c + 1 < n_chunks:
                hs[c + 1] = pltpu.async_copy(
                    table_hbm.at[idx_v.at[pl.ds((c + 1) * chunk, chunk)]],
                    bufs[(c + 1) % 2], sems[(c + 1) % 2])
            hs[c].wait()
            pltpu.sync_copy(bufs[c % 2],
                            out_hbm.at[pl.ds(base + c * chunk, chunk)])

    return k(table, idx)


BF = FFN // 8
NF = FFN // BF


def _group_body(meta_ref, xs_ref, ws_ref, w1_ref, w3_ref, w2_ref, ys_ref,
                acc_ref):
    f = pl.program_id(0)
    b = pl.program_id(1)

    @pl.when(b < meta_ref[NB])
    def _():
        x = xs_ref[...]
        h1 = jnp.dot(x, w1_ref[0], preferred_element_type=jnp.float32)
        h3 = jnp.dot(x, w3_ref[0], preferred_element_type=jnp.float32)
        p = h1 * (1.0 / (1.0 + jnp.exp(-h1))) * h3
        po = jnp.dot(p, w2_ref[0], preferred_element_type=jnp.float32)
        sl = pl.ds(b * BT, BT)

        @pl.when(f == 0)
        def _():
            acc_ref[sl, :] = po

        @pl.when(jnp.logical_and(f > 0, f < NF - 1))
        def _():
            acc_ref[sl, :] += po

        @pl.when(f == NF - 1)
        def _():
            ys_ref[...] = (acc_ref[sl, :] + po) * ws_ref[...]


def _add_body(ya_ref, yb_ref, o_ref):
    o_ref[...] = ya_ref[...] + yb_ref[...]


def kernel(hidden_states, Wg, W1, W3, W2):
    x = hidden_states.reshape(S, D)
    wfull, selbuf, rwbuf, auxbuf = pl.pallas_call(
        _router_body,
        out_shape=[
            jax.ShapeDtypeStruct((S, E), jnp.float32),
            jax.ShapeDtypeStruct((S, E), jnp.int32),
            jax.ShapeDtypeStruct((S, E), jnp.float32),
            jax.ShapeDtypeStruct((8, 128), jnp.float32),
        ],
    )(x, Wg)

    keys = jnp.concatenate([selbuf[:, 0], selbuf[:, 1]])
    wvals = jnp.concatenate([rwbuf[:, 0], rwbuf[:, 1]])
    tok = jnp.tile(jnp.arange(S, dtype=jnp.int32), 2)
    order = jnp.argsort(keys, stable=True)
    inv = jnp.argsort(order).astype(jnp.int32)
    counts = jnp.bincount(keys, length=E).astype(jnp.int32)
    pad_counts = ((counts + BT - 1) // BT) * BT
    css = jnp.concatenate([jnp.zeros(1, jnp.int32), jnp.cumsum(counts)])[:E]
    csp = jnp.concatenate([jnp.zeros(1, jnp.int32), jnp.cumsum(pad_counts)])[:E]
    ppos = (csp[keys] + (inv - css[keys])).astype(jnp.int32)
    gidx = jnp.zeros((BP,), jnp.int32).at[ppos].set(tok)
    ws = jnp.zeros((BP, 1), jnp.float32).at[ppos, 0].set(wvals)
    nb_actual = (jnp.sum(pad_counts) // BT).astype(jnp.int32)
    starts_blocks = csp // BT
    bid = jnp.arange(NB, dtype=jnp.int32)
    be = jnp.sum((bid[:, None] >= starts_blocks[None, :]).astype(jnp.int32),
                 axis=1) - 1
    tail_e = be[jnp.clip(nb_actual - 1, 0, NB - 1)]
    be = jnp.where(bid < nb_actual, be, tail_e)
    meta = jnp.concatenate([be, nb_actual[None]]).astype(jnp.int32)

    xs = _sc_gather(x, gidx, BP, 4)

    grid_spec = pltpu.PrefetchScalarGridSpec(
        num_scalar_prefetch=1,
        grid=(NF, NB),
        in_specs=[
            pl.BlockSpec((BT, D), lambda f, b, m: (b, 0)),
            pl.BlockSpec((BT, 1), lambda f, b, m: (b, 0)),
            pl.BlockSpec((1, D, BF), lambda f, b, m: (m[b], 0, f)),
            pl.BlockSpec((1, D, BF), lambda f, b, m: (m[b], 0, f)),
            pl.BlockSpec((1, BF, D), lambda f, b, m: (m[b], f, 0)),
        ],
        out_specs=pl.BlockSpec(
            (BT, D), lambda f, b, m: (jnp.where(f == NF - 1, b, 0), 0)),
        scratch_shapes=[pltpu.VMEM((BP, D), jnp.float32)],
    )
    ys = pl.pallas_call(
        _group_body,
        grid_spec=grid_spec,
        out_shape=jax.ShapeDtypeStruct((BP, D), jnp.float32),
    )(meta, xs, ws, W1, W3, W2)

    ya = _sc_gather(ys, ppos, NA, 4)
    out = pl.pallas_call(
        _add_body,
        grid=(8,),
        in_specs=[
            pl.BlockSpec((S // 8, D), lambda i: (i, 0)),
            pl.BlockSpec((S // 8, D), lambda i: (i + 8, 0)),
        ],
        out_specs=pl.BlockSpec((S // 8, D), lambda i: (i, 0)),
        out_shape=jax.ShapeDtypeStruct((S, D), jnp.float32),
    )(ya, ya)

    return out.reshape(1, S, D), auxbuf[0, 0]


# speedup vs baseline: 1.2468x; 1.2468x over previous
import functools

import jax
import jax.numpy as jnp
from jax import lax
from jax.experimental import pallas as pl
from jax.experimental.pallas import tpu as pltpu
from jax.experimental.pallas import tpu_sc as plsc

S = 2048
D = 1024
FFN = 4096
E = 8
BT = 128
NB = (2 * S + E * (BT - 1) + BT - 1) // BT
BP = NB * BT
NA = 2 * S


def _router_body(x_ref, wg_ref, sel_ref, rw_ref, aux_ref):
    x = x_ref[...]
    wg = wg_ref[...]
    logits = jnp.dot(x, wg, preferred_element_type=jnp.float32)
    pos = jax.lax.broadcasted_iota(jnp.int32, (S, E), 1)
    m1 = jnp.max(logits, axis=1, keepdims=True)
    p1 = jnp.min(jnp.where(logits == m1, pos, E), axis=1, keepdims=True)
    oh1 = pos == p1
    l2 = jnp.where(oh1, -jnp.inf, logits)
    m2 = jnp.max(l2, axis=1, keepdims=True)
    p2 = jnp.min(jnp.where(l2 == m2, pos, E), axis=1, keepdims=True)
    oh2 = pos == p2
    ed = jnp.exp(m2 - m1)
    w1 = 1.0 / (1.0 + ed)
    w2 = ed / (1.0 + ed)
    zi = jnp.zeros((S, E - 2), jnp.int32)
    zf = jnp.zeros((S, E - 2), jnp.float32)
    sel_ref[...] = jnp.concatenate([p1, p2, zi], axis=1)
    rw_ref[...] = jnp.concatenate([w1, w2, zf], axis=1)
    sm = jnp.exp(logits - m1)
    sm = sm / jnp.sum(sm, axis=1, keepdims=True)
    prob = jnp.sum(sm, axis=0, keepdims=True) * (1.0 / S)
    tp = (jnp.sum(oh1.astype(jnp.float32), axis=0, keepdims=True)
          + jnp.sum(oh2.astype(jnp.float32), axis=0, keepdims=True)) * (1.0 / S)
    aux = jnp.sum(tp * prob) * E
    aux_ref[...] = jnp.full((8, 128), aux, jnp.float32)


def _sc_scatter_rows(src, idx3):
    info = plsc.get_sparse_core_info()
    nw = info.num_cores * info.num_subcores
    n_chunks, chunk = idx3.shape[1], idx3.shape[2]
    per_w = n_chunks * chunk
    ncols = src.shape[1]
    mesh = plsc.VectorSubcoreMesh(core_axis_name="c", subcore_axis_name="s")

    @functools.partial(
        pl.kernel, mesh=mesh,
        out_type=jax.ShapeDtypeStruct((BP, ncols), jnp.float32),
        scratch_types=[
            pltpu.VMEM((n_chunks, chunk), jnp.int32),
            pltpu.VMEM((chunk, ncols), jnp.float32),
            pltpu.VMEM((chunk, ncols), jnp.float32),
            pltpu.SemaphoreType.DMA,
            pltpu.SemaphoreType.DMA,
        ],
    )
    def k(src_hbm, idx_hbm, out_hbm, idx_v, rows0, rows1, sem0, sem1):
        wid = lax.axis_index("s") * info.num_cores + lax.axis_index("c")
        src_base = (wid % (nw // 2)) * per_w
        bufs = (rows0, rows1)
        sems = (sem0, sem1)
        pltpu.sync_copy(idx_hbm.at[wid], idx_v)
        hs = [None] * n_chunks
        for c in range(n_chunks):
            if c >= 2:
                hs[c - 2].wait()
            pltpu.sync_copy(src_hbm.at[pl.ds(src_base + c * chunk, chunk)],
                            bufs[c % 2])
            hs[c] = pltpu.async_copy(bufs[c % 2], out_hbm.at[idx_v.at[c]],
                                     sems[c % 2])
        for c in range(max(0, n_chunks - 2), n_chunks):
            hs[c].wait()

    return k(src, idx3)


def _sc_gather(table, idx, n_rows, n_chunks):
    info = plsc.get_sparse_core_info()
    nw = info.num_cores * info.num_subcores
    per_w = n_rows // nw
    chunk = per_w // n_chunks
    ncols = table.shape[1]
    mesh = plsc.VectorSubcoreMesh(core_axis_name="c", subcore_axis_name="s")

    @functools.partial(
        pl.kernel, mesh=mesh,
        out_type=jax.ShapeDtypeStruct((n_rows, ncols), table.dtype),
        scratch_types=[
            pltpu.VMEM((per_w,), jnp.int32),
            pltpu.VMEM((chunk, ncols), table.dtype),
            pltpu.VMEM((chunk, ncols), table.dtype),
            pltpu.SemaphoreType.DMA,
            pltpu.SemaphoreType.DMA,
        ],
    )
    def k(table_hbm, idx_hbm, out_hbm, idx_v, rows0, rows1, sem0, sem1):
        wid = lax.axis_index("s") * info.num_cores + lax.axis_index("c")
        base = wid * per_w
        bufs = (rows0, rows1)
        sems = (sem0, sem1)
        pltpu.sync_copy(idx_hbm.at[pl.ds(base, per_w)], idx_v)
        hs = [None] * n_chunks
        hs[0] = pltpu.async_copy(
            table_hbm.at[idx_v.at[pl.ds(0, chunk)]], bufs[0], sems[0])
        for c in range(n_chunks):
            if c + 1 < n_chunks:
                hs[c + 1] = pltpu.async_copy(
                    table_hbm.at[idx_v.at[pl.ds((c + 1) * chunk, chunk)]],
                    bufs[(c + 1) % 2], sems[(c + 1) % 2])
            hs[c].wait()
            pltpu.sync_copy(bufs[c % 2],
                            out_hbm.at[pl.ds(base + c * chunk, chunk)])

    return k(table, idx)


def _mlp_in_body(meta_ref, xs_ref, w1_ref, w3_ref, h_ref):
    b = pl.program_id(0)

    @pl.when(b < meta_ref[NB])
    def _():
        x = xs_ref[...].astype(jnp.bfloat16)
        h1 = jnp.dot(x, w1_ref[0], preferred_element_type=jnp.float32)
        h3 = jnp.dot(x, w3_ref[0], preferred_element_type=jnp.float32)
        h_ref[...] = (h1 * (1.0 / (1.0 + jnp.exp(-h1))) * h3
                      ).astype(jnp.bfloat16)


def _mlp_out_body(meta_ref, h_ref, w2_ref, ys_ref):
    b = pl.program_id(0)

    @pl.when(b < meta_ref[NB])
    def _():
        ys_ref[...] = jnp.dot(h_ref[...], w2_ref[0],
                              preferred_element_type=jnp.float32)


def _combine_body(ya_ref, yb_ref, wa_ref, wb_ref, o_ref):
    o_ref[...] = wa_ref[...] * ya_ref[...] + wb_ref[...] * yb_ref[...]


def kernel(hidden_states, Wg, W1, W3, W2):
    x = hidden_states.reshape(S, D)
    selbuf, rwbuf, auxbuf = pl.pallas_call(
        _router_body,
        out_shape=[
            jax.ShapeDtypeStruct((S, E), jnp.int32),
            jax.ShapeDtypeStruct((S, E), jnp.float32),
            jax.ShapeDtypeStruct((8, 128), jnp.float32),
        ],
    )(x, Wg)

    keys = jnp.concatenate([selbuf[:, 0], selbuf[:, 1]])
    order = jnp.argsort(keys, stable=True)
    inv = jnp.argsort(order).astype(jnp.int32)
    counts = jnp.bincount(keys, length=E).astype(jnp.int32)
    pad_counts = ((counts + BT - 1) // BT) * BT
    css = jnp.concatenate([jnp.zeros(1, jnp.int32), jnp.cumsum(counts)])[:E]
    csp = jnp.concatenate([jnp.zeros(1, jnp.int32), jnp.cumsum(pad_counts)])[:E]
    ppos = (csp[keys] + (inv - css[keys])).astype(jnp.int32)
    nb_actual = (jnp.sum(pad_counts) // BT).astype(jnp.int32)
    starts_blocks = csp // BT
    bid = jnp.arange(NB, dtype=jnp.int32)
    be = jnp.sum((bid[:, None] >= starts_blocks[None, :]).astype(jnp.int32),
                 axis=1) - 1
    tail_e = be[jnp.clip(nb_actual - 1, 0, NB - 1)]
    be = jnp.where(bid < nb_actual, be, tail_e)
    meta = jnp.concatenate([be, nb_actual[None]]).astype(jnp.int32)

    xs = _sc_scatter_rows(x, ppos.reshape(32, 4, 32))

    w1b = W1.astype(jnp.bfloat16)
    w3b = W3.astype(jnp.bfloat16)
    w2b = W2.astype(jnp.bfloat16)
    spec_a = pltpu.PrefetchScalarGridSpec(
        num_scalar_prefetch=1,
        grid=(NB,),
        in_specs=[
            pl.BlockSpec((BT, D), lambda b, m: (b, 0)),
            pl.BlockSpec((1, D, FFN), lambda b, m: (m[b], 0, 0)),
            pl.BlockSpec((1, D, FFN), lambda b, m: (m[b], 0, 0)),
        ],
        out_specs=pl.BlockSpec((BT, FFN), lambda b, m: (b, 0)),
    )
    h = pl.pallas_call(
        _mlp_in_body,
        grid_spec=spec_a,
        out_shape=jax.ShapeDtypeStruct((BP, FFN), jnp.bfloat16),
    )(meta, xs, w1b, w3b)

    spec_b = pltpu.PrefetchScalarGridSpec(
        num_scalar_prefetch=1,
        grid=(NB,),
        in_specs=[
            pl.BlockSpec((BT, FFN), lambda b, m: (b, 0)),
            pl.BlockSpec((1, FFN, D), lambda b, m: (m[b], 0, 0)),
        ],
        out_specs=pl.BlockSpec((BT, D), lambda b, m: (b, 0)),
    )
    ys = pl.pallas_call(
        _mlp_out_body,
        grid_spec=spec_b,
        out_shape=jax.ShapeDtypeStruct((BP, D), jnp.float32),
    )(meta, h, w2b)

    ya = _sc_gather(ys, ppos, NA, 4)
    wa = rwbuf[:, 0:1]
    wb = rwbuf[:, 1:2]
    nblk = 8
    out = pl.pallas_call(
        _combine_body,
        grid=(nblk,),
        in_specs=[
            pl.BlockSpec((S // nblk, D), lambda i: (i, 0)),
            pl.BlockSpec((S // nblk, D), lambda i: (i + nblk, 0)),
            pl.BlockSpec((S // nblk, 1), lambda i: (i, 0)),
            pl.BlockSpec((S // nblk, 1), lambda i: (i, 0)),
        ],
        out_specs=pl.BlockSpec((S // nblk, D), lambda i: (i, 0)),
        out_shape=jax.ShapeDtypeStruct((S, D), jnp.float32),
    )(ya, ya, wa, wb)

    return out.reshape(1, S, D), auxbuf[0, 0]
